# baseline (speedup 1.0000x reference)
import jax
import jax.numpy as jnp
import numpy as np
from jax import lax
from jax.experimental import pallas as pl
from jax.experimental.pallas import tpu as pltpu
from jax.experimental.pallas import tpu_sc as plsc

_PAT = np.array(sorted([1, 4, 8, 11, 15, 19, 22, 26, 30, 33, 37, 41, 44,
                        48, 52, 55, 59, 63, 66, 70, 74, 77, 81, 85, 88, 92]),
                dtype=np.int32)

_IN_W = 100
_OUT_W = 26
_R = 1024 * 50 * 16
_NC = 2
_NS = 16
_NW = _NC * _NS
_ROWS_PER_W = _R // _NW
_C = 256
_CHUNKS = _ROWS_PER_W // _C
_GRPS = _C * _OUT_W // 16


def _body(in_hbm, idx_hbm, out_hbm, in_v, idx_v, out_v):
    cid = lax.axis_index("c")
    sid = lax.axis_index("s")
    wid = sid * _NC + cid
    base_row = wid * _ROWS_PER_W

    pltpu.sync_copy(idx_hbm, idx_v)

    def chunk_body(ci, carry):
        row0 = base_row + ci * _C
        pltpu.sync_copy(in_hbm.at[pl.ds(row0 * _IN_W, _C * _IN_W)], in_v)

        def grp(v, c2):
            iv = idx_v[pl.ds(v * 16, 16)]
            out_v[pl.ds(v * 16, 16)] = plsc.load_gather(in_v, [iv])
            return c2

        lax.fori_loop(0, _GRPS, grp, 0)
        pltpu.sync_copy(out_v, out_hbm.at[pl.ds(row0 * _OUT_W, _C * _OUT_W)])
        return carry

    lax.fori_loop(0, _CHUNKS, chunk_body, 0)


@jax.jit
def _run(flat_in, idx_table):
    fn = pl.kernel(
        _body,
        out_type=jax.ShapeDtypeStruct((_R * _OUT_W,), jnp.float32),
        mesh=plsc.VectorSubcoreMesh(core_axis_name="c", subcore_axis_name="s"),
        scratch_types=[
            pltpu.VMEM((_C * _IN_W,), jnp.float32),
            pltpu.VMEM((_C * _OUT_W,), jnp.int32),
            pltpu.VMEM((_C * _OUT_W,), jnp.float32),
        ],
        compiler_params=pltpu.CompilerParams(needs_layout_passes=False),
    )
    return fn(flat_in, idx_table)


def kernel(inputs):
    o = np.arange(_C * _OUT_W, dtype=np.int32)
    idx_table = jnp.asarray((o // _OUT_W) * _IN_W + _PAT[o % _OUT_W])
    flat = inputs.reshape(-1)
    out = _run(flat, idx_table)
    return out.reshape(1024, 50, 16, _OUT_W)


# speedup vs baseline: 1.1621x; 1.1621x over previous
import jax
import jax.numpy as jnp
import numpy as np
from jax import lax
from jax.experimental import pallas as pl
from jax.experimental.pallas import tpu as pltpu
from jax.experimental.pallas import tpu_sc as plsc

_PAT = np.array(sorted([1, 4, 8, 11, 15, 19, 22, 26, 30, 33, 37, 41, 44,
                        48, 52, 55, 59, 63, 66, 70, 74, 77, 81, 85, 88, 92]),
                dtype=np.int32)

_IN_W = 100
_OUT_W = 26
_R = 1024 * 50 * 16
_NC = 2
_NS = 16
_NW = _NC * _NS
_ROWS_PER_W = _R // _NW
_C = 256
_CHUNKS = _ROWS_PER_W // _C
_GB = 8
_NVEC = _GB * _OUT_W // 16
_BLKS = _C // _GB


def _body(in_hbm, idx_hbm, out_hbm,
          in_v0, in_v1, out_v0, out_v1, idx_v,
          sin0, sin1, sout0, sout1):
    cid = lax.axis_index("c")
    sid = lax.axis_index("s")
    wid = sid * _NC + cid
    base_row = wid * _ROWS_PER_W

    pltpu.sync_copy(idx_hbm, idx_v)
    idx13 = [idx_v[pl.ds(v * 16, 16)] for v in range(_NVEC)]

    in_bufs = [in_v0, in_v1]
    out_bufs = [out_v0, out_v1]
    sins = [sin0, sin1]
    souts = [sout0, sout1]

    def in_src(c):
        return in_hbm.at[pl.ds((base_row + c * _C) * _IN_W, _C * _IN_W)]

    def out_dst(c):
        return out_hbm.at[pl.ds((base_row + c * _C) * _OUT_W, _C * _OUT_W)]

    pltpu.async_copy(in_src(0), in_v0, sin0)
    pltpu.async_copy(in_src(1), in_v1, sin1)

    @pl.loop(0, _CHUNKS, step=2)
    def chunk_loop(ci):
        for b in range(2):
            c = ci + b
            pltpu.make_async_copy(in_src(c), in_bufs[b], sins[b]).wait()

            @pl.when(c >= 2)
            def _wait_out():
                pltpu.make_async_copy(
                    out_bufs[b], out_dst(c - 2), souts[b]).wait()

            @pl.loop(0, _BLKS)
            def blk(g):
                ib = g * (_GB * _IN_W)
                ob = g * (_GB * _OUT_W)
                for v in range(_NVEC):
                    iv = idx13[v] + ib
                    out_bufs[b][pl.ds(ob + v * 16, 16)] = (
                        plsc.load_gather(in_bufs[b], [iv]))

            pltpu.async_copy(out_bufs[b], out_dst(c), souts[b])

            @pl.when(c + 2 < _CHUNKS)
            def _next_in():
                pltpu.async_copy(in_src(c + 2), in_bufs[b], sins[b])

    pltpu.make_async_copy(out_bufs[0], out_dst(_CHUNKS - 2), souts[0]).wait()
    pltpu.make_async_copy(out_bufs[1], out_dst(_CHUNKS - 1), souts[1]).wait()


@jax.jit
def _run(flat_in, idx_table):
    fn = pl.kernel(
        _body,
        out_type=jax.ShapeDtypeStruct((_R * _OUT_W,), jnp.float32),
        mesh=plsc.VectorSubcoreMesh(core_axis_name="c", subcore_axis_name="s"),
        scratch_types=[
            pltpu.VMEM((_C * _IN_W,), jnp.float32),
            pltpu.VMEM((_C * _IN_W,), jnp.float32),
            pltpu.VMEM((_C * _OUT_W,), jnp.float32),
            pltpu.VMEM((_C * _OUT_W,), jnp.float32),
            pltpu.VMEM((_GB * _OUT_W,), jnp.int32),
            pltpu.SemaphoreType.DMA,
            pltpu.SemaphoreType.DMA,
            pltpu.SemaphoreType.DMA,
            pltpu.SemaphoreType.DMA,
        ],
        compiler_params=pltpu.CompilerParams(needs_layout_passes=False),
    )
    return fn(flat_in, idx_table)


def kernel(inputs):
    o = np.arange(_GB * _OUT_W, dtype=np.int32)
    idx_table = jnp.asarray((o // _OUT_W) * _IN_W + _PAT[o % _OUT_W])
    flat = inputs.reshape(-1)
    out = _run(flat, idx_table)
    return out.reshape(1024, 50, 16, _OUT_W)


# speedup vs baseline: 1.1849x; 1.0197x over previous
import jax
import jax.numpy as jnp
import numpy as np
from jax import lax
from jax.experimental import pallas as pl
from jax.experimental.pallas import tpu as pltpu
from jax.experimental.pallas import tpu_sc as plsc

_PAT = np.array(sorted([1, 4, 8, 11, 15, 19, 22, 26, 30, 33, 37, 41, 44,
                        48, 52, 55, 59, 63, 66, 70, 74, 77, 81, 85, 88, 92]),
                dtype=np.int32)

_IN_W = 100
_OUT_W = 26
_R = 1024 * 50 * 16
_NC = 2
_NS = 16
_NW = _NC * _NS
_ROWS_PER_W = _R // _NW
_C = 256
_CHUNKS = _ROWS_PER_W // _C
_GB = 8
_NVEC = _GB * _OUT_W // 16
_BLKS = _C // _GB


def _body(in_hbm, idx_hbm, out_hbm,
          in_v0, in_v1, out_v0, out_v1, idx_v,
          sin0, sin1, sout0, sout1):
    cid = lax.axis_index("c")
    sid = lax.axis_index("s")
    wid = sid * _NC + cid
    base_row = wid * _ROWS_PER_W

    pltpu.sync_copy(idx_hbm, idx_v)
    idx13 = [idx_v[pl.ds(v * 16, 16)] for v in range(_NVEC)]

    in_bufs = [in_v0, in_v1]
    out_bufs = [out_v0, out_v1]
    sins = [sin0, sin1]
    souts = [sout0, sout1]

    def in_src(c):
        return in_hbm.at[pl.ds((base_row + c * _C) * _IN_W, _C * _IN_W)]

    def out_dst(c):
        return out_hbm.at[pl.ds((base_row + c * _C) * _OUT_W, _C * _OUT_W)]

    pltpu.async_copy(in_src(0), in_v0, sin0)
    pltpu.async_copy(in_src(1), in_v1, sin1)

    @pl.loop(0, _CHUNKS, step=2)
    def chunk_loop(ci):
        for b in range(2):
            c = ci + b
            pltpu.make_async_copy(in_src(c), in_bufs[b], sins[b]).wait()

            @pl.when(c >= 2)
            def _wait_out():
                pltpu.make_async_copy(
                    out_bufs[b], out_dst(c - 2), souts[b]).wait()

            @plsc.parallel_loop(0, _BLKS, unroll=2)
            def blk(g):
                ib = g * (_GB * _IN_W)
                ob = g * (_GB * _OUT_W)
                for v in range(_NVEC):
                    iv = idx13[v] + ib
                    out_bufs[b][pl.ds(ob + v * 16, 16)] = (
                        plsc.load_gather(in_bufs[b], [iv]))

            pltpu.async_copy(out_bufs[b], out_dst(c), souts[b])

            @pl.when(c + 2 < _CHUNKS)
            def _next_in():
                pltpu.async_copy(in_src(c + 2), in_bufs[b], sins[b])

    pltpu.make_async_copy(out_bufs[0], out_dst(_CHUNKS - 2), souts[0]).wait()
    pltpu.make_async_copy(out_bufs[1], out_dst(_CHUNKS - 1), souts[1]).wait()


@jax.jit
def _run(flat_in, idx_table):
    fn = pl.kernel(
        _body,
        out_type=jax.ShapeDtypeStruct((_R * _OUT_W,), jnp.float32),
        mesh=plsc.VectorSubcoreMesh(core_axis_name="c", subcore_axis_name="s"),
        scratch_types=[
            pltpu.VMEM((_C * _IN_W,), jnp.float32),
            pltpu.VMEM((_C * _IN_W,), jnp.float32),
            pltpu.VMEM((_C * _OUT_W,), jnp.float32),
            pltpu.VMEM((_C * _OUT_W,), jnp.float32),
            pltpu.VMEM((_GB * _OUT_W,), jnp.int32),
            pltpu.SemaphoreType.DMA,
            pltpu.SemaphoreType.DMA,
            pltpu.SemaphoreType.DMA,
            pltpu.SemaphoreType.DMA,
        ],
        compiler_params=pltpu.CompilerParams(needs_layout_passes=False),
    )
    return fn(flat_in, idx_table)


def kernel(inputs):
    o = np.arange(_GB * _OUT_W, dtype=np.int32)
    idx_table = jnp.asarray((o // _OUT_W) * _IN_W + _PAT[o % _OUT_W])
    flat = inputs.reshape(-1)
    out = _run(flat, idx_table)
    return out.reshape(1024, 50, 16, _OUT_W)


# speedup vs baseline: 2.1550x; 1.8186x over previous
import jax
import jax.numpy as jnp
import numpy as np
from jax import lax
from jax.experimental import pallas as pl
from jax.experimental.pallas import tpu as pltpu
from jax.experimental.pallas import tpu_sc as plsc

_PAT = np.array(sorted([1, 4, 8, 11, 15, 19, 22, 26, 30, 33, 37, 41, 44,
                        48, 52, 55, 59, 63, 66, 70, 74, 77, 81, 85, 88, 92]),
                dtype=np.int32)

_B = 1024
_T = 50
_H = 16
_IN_W = 100
_OUT_W = 26
_NC = 2
_NS = 16
_NW = _NC * _NS
_BPW = _B // _NW
_TC = 10
_CPB = _T // _TC
_CHUNKS = _BPW * _CPB
_NVEC = _H * _OUT_W // 16


def _body(in_hbm, idx_hbm, out_hbm,
          in_v0, in_v1, out_v0, out_v1, idx_v,
          sin0, sin1, sout0, sout1):
    cid = lax.axis_index("c")
    sid = lax.axis_index("s")
    wid = sid * _NC + cid
    base_b = wid * _BPW

    pltpu.sync_copy(idx_hbm, idx_v)
    n = _NVEC * 16
    hv = [idx_v[pl.ds(0 * n + v * 16, 16)] for v in range(_NVEC)]
    cv = [idx_v[pl.ds(1 * n + v * 16, 16)] for v in range(_NVEC)]
    ov = [idx_v[pl.ds(2 * n + v * 16, 16)] for v in range(_NVEC)]
    zero = idx_v[pl.ds(3 * n, 16)]

    in_bufs = [in_v0, in_v1]
    out_bufs = [out_v0, out_v1]
    sins = [sin0, sin1]
    souts = [sout0, sout1]

    def in_src(c):
        return in_hbm.at[base_b + c // _CPB, pl.ds((c % _CPB) * _TC, _TC)]

    def out_dst(c):
        return out_hbm.at[base_b + c // _CPB, pl.ds((c % _CPB) * _TC, _TC)]

    pltpu.async_copy(in_src(0), in_v0, sin0)
    pltpu.async_copy(in_src(1), in_v1, sin1)

    @pl.loop(0, _CHUNKS, step=2)
    def chunk_loop(cc):
        for b in range(2):
            c = cc + b
            pltpu.make_async_copy(in_src(c), in_bufs[b], sins[b]).wait()

            @pl.when(c >= 2)
            def _wait_out():
                pltpu.make_async_copy(
                    out_bufs[b], out_dst(c - 2), souts[b]).wait()

            @plsc.parallel_loop(0, _TC)
            def slab(a):
                av = zero + a
                for v in range(_NVEC):
                    x = plsc.load_gather(in_bufs[b], [av, hv[v], cv[v]])
                    plsc.store_scatter(out_bufs[b], [av, hv[v], ov[v]], x)

            pltpu.async_copy(out_bufs[b], out_dst(c), souts[b])

            @pl.when(c + 2 < _CHUNKS)
            def _next_in():
                pltpu.async_copy(in_src(c + 2), in_bufs[b], sins[b])

    pltpu.make_async_copy(out_bufs[0], out_dst(_CHUNKS - 2), souts[0]).wait()
    pltpu.make_async_copy(out_bufs[1], out_dst(_CHUNKS - 1), souts[1]).wait()


@jax.jit
def _run(in4, idx_table):
    fn = pl.kernel(
        _body,
        out_type=jax.ShapeDtypeStruct((_B, _T, _H, _OUT_W), jnp.float32),
        mesh=plsc.VectorSubcoreMesh(core_axis_name="c", subcore_axis_name="s"),
        scratch_types=[
            pltpu.VMEM((_TC, _H, _IN_W), jnp.float32),
            pltpu.VMEM((_TC, _H, _IN_W), jnp.float32),
            pltpu.VMEM((_TC, _H, _OUT_W), jnp.float32),
            pltpu.VMEM((_TC, _H, _OUT_W), jnp.float32),
            pltpu.VMEM((3 * _NVEC * 16 + 16,), jnp.int32),
            pltpu.SemaphoreType.DMA,
            pltpu.SemaphoreType.DMA,
            pltpu.SemaphoreType.DMA,
            pltpu.SemaphoreType.DMA,
        ],
        compiler_params=pltpu.CompilerParams(needs_layout_passes=False),
    )
    return fn(in4, idx_table)


def kernel(inputs):
    o = np.arange(_H * _OUT_W, dtype=np.int32)
    rows = o // _OUT_W
    cols_in = _PAT[o % _OUT_W]
    cols_out = o % _OUT_W
    zeros = np.zeros(16, dtype=np.int32)
    idx_table = jnp.asarray(np.concatenate([rows, cols_in, cols_out, zeros]))
    return _run(inputs, idx_table)


# speedup vs baseline: 25.5067x; 11.8361x over previous
import jax
import jax.numpy as jnp
import numpy as np
from jax import lax
from jax.experimental import pallas as pl
from jax.experimental.pallas import tpu as pltpu
from jax.experimental.pallas import tpu_sc as plsc

_PAT = np.array(sorted([1, 4, 8, 11, 15, 19, 22, 26, 30, 33, 37, 41, 44,
                        48, 52, 55, 59, 63, 66, 70, 74, 77, 81, 85, 88, 92]),
                dtype=np.int32)

_B = 1024
_T = 50
_H = 16
_IN_W = 100
_OUT_W = 26
_NC = 2
_NS = 16
_NW = _NC * _NS
_P = _T * _OUT_W
_K = (_P + _NW - 1) // _NW
_NBUF = 6
_LOOK = 3
_TBL = 1344


def _body(in_hbm4, tbl_hbm, out_hbm4,
          buf0, buf1, buf2, buf3, buf4, buf5, tbl_v,
          si0, si1, si2, si3, si4, si5,
          so0, so1, so2, so3, so4, so5):
    tin = in_hbm4.reshape(_T * _IN_W, _H, _B)
    tout = out_hbm4.reshape(_P, _H, _B)
    bufs = [buf0, buf1, buf2, buf3, buf4, buf5]
    sins = [si0, si1, si2, si3, si4, si5]
    souts = [so0, so1, so2, so3, so4, so5]

    cid = lax.axis_index("c")
    sid = lax.axis_index("s")
    wid = sid * _NC + cid

    pltpu.sync_copy(tbl_hbm, tbl_v)

    def src_of(k):
        v = tbl_v[pl.ds(wid + k * _NW, 16)]
        return v[0]

    def pair_of(k):
        return wid + k * _NW

    def start_in(k, b):
        pltpu.async_copy(tin.at[src_of(k)], bufs[b], sins[b])

    def wait_in(k, b):
        pltpu.make_async_copy(tin.at[src_of(k)], bufs[b], sins[b]).wait()

    def start_out(k, b):
        pltpu.async_copy(bufs[b], tout.at[pair_of(k)], souts[b])

    def wait_out(k, b):
        pltpu.make_async_copy(bufs[b], tout.at[pair_of(k)], souts[b]).wait()

    for k0 in range(_LOOK):
        @pl.when(pair_of(k0) < _P)
        def _prime():
            start_in(k0, k0 % _NBUF)

    _NG = -(-_K // _NBUF)

    @pl.loop(0, _NG)
    def group(g):
        for i in range(_NBUF):
            k = g * _NBUF + i

            @pl.when(pair_of(k) < _P)
            def _do():
                wait_in(k, i)
                start_out(k, i)

                @pl.when(pair_of(k + _LOOK) < _P)
                def _ahead():
                    start_in(k + _LOOK, (i + _LOOK) % _NBUF)

            @pl.when((k >= _LOOK) & (pair_of(k - _LOOK) < _P))
            def _drain():
                wait_out(k - _LOOK, (i - _LOOK) % _NBUF)

    for kk in range(_NG * _NBUF - _LOOK, _K):
        @pl.when(pair_of(kk) < _P)
        def _tail():
            wait_out(kk, kk % _NBUF)


@jax.jit
def _run(tin4, tbl):
    fn = pl.kernel(
        _body,
        out_type=jax.ShapeDtypeStruct((_T, _OUT_W, _H, _B), jnp.float32),
        mesh=plsc.VectorSubcoreMesh(core_axis_name="c", subcore_axis_name="s"),
        scratch_types=(
            [pltpu.VMEM((_H, _B), jnp.float32) for _ in range(_NBUF)]
            + [pltpu.VMEM((_TBL,), jnp.int32)]
            + [pltpu.SemaphoreType.DMA for _ in range(2 * _NBUF)]
        ),
        compiler_params=pltpu.CompilerParams(needs_layout_passes=False),
    )
    return fn(tin4, tbl)


def kernel(inputs):
    p = np.arange(_TBL, dtype=np.int64)
    pc = np.minimum(p, _P - 1)
    tbl = ((pc // _OUT_W) * _IN_W + _PAT[pc % _OUT_W]).astype(np.int32)
    tin = jnp.transpose(inputs, (1, 3, 2, 0))
    tout = _run(tin, jnp.asarray(tbl))
    return jnp.transpose(tout, (3, 0, 2, 1))
